# baseline (speedup 1.0000x reference)
import functools

import numpy as np
import jax
import jax.numpy as jnp
from jax.experimental import pallas as pl
from jax.experimental.pallas import tpu as pltpu

_L = 128
_GROUPS = 128


@functools.lru_cache(maxsize=None)
def _flat_constants(hw: int):
    j = np.arange(_L)
    s0 = (hw * j) // _L
    r0 = (hw * j) % _L
    spill = np.maximum(0, r0 + hw - _L)

    l = np.arange(_L)[:, None]
    c0 = ((l >= r0[None, :]) & (l < np.minimum(_L, r0 + hw)[None, :]))
    c1 = (l < spill[None, :])
    cc = np.concatenate([c0, c1], axis=1).astype(np.float32)

    S = np.arange(hw * _GROUPS)[:, None] % hw
    m0 = (S == s0[None, :]).astype(np.float32)
    m1 = (S == (s0 + 1)[None, :]).astype(np.float32)

    a = np.arange(_GROUPS)[:, None]
    Sg = np.arange(hw * _GROUPS)[None, :] // hw
    lmat = (Sg == a).astype(np.float32)
    return cc, m0, m1, lmat


def _flat_body(x_ref, cc_ref, m0_ref, m1_ref, l_ref, o_ref, *,
               n_chunks, chunk, inv_hw):
    acc = jnp.zeros((_GROUPS, _L), jnp.float32)
    cc = cc_ref[...]
    dn = (((1,), (0,)), ((), ()))
    for i in range(n_chunks):
        lo, hi = i * chunk, (i + 1) * chunk
        v = x_ref[lo:hi, :].astype(jnp.bfloat16)
        x01 = jax.lax.dot_general(v, cc, dn,
                                  preferred_element_type=jnp.float32)
        y = (x01[:, :_L] * m0_ref[lo:hi, :]
             + x01[:, _L:] * m1_ref[lo:hi, :])
        yhi = y.astype(jnp.bfloat16)
        ylo = (y - yhi.astype(jnp.float32)).astype(jnp.bfloat16)
        lc = l_ref[:, lo:hi]
        acc = acc + jax.lax.dot_general(lc, yhi, dn,
                                        preferred_element_type=jnp.float32)
        acc = acc + jax.lax.dot_general(lc, ylo, dn,
                                        preferred_element_type=jnp.float32)
    o_ref[...] = acc * inv_hw


def _gap_flat(x, N, C, hw):
    rows = N * C
    inv_hw = 1.0 / float(hw)
    xf = x.reshape(rows * hw // _L, _L)

    cc, m0, m1, lmat = _flat_constants(hw)
    block_s = hw * _GROUPS
    grid = (xf.shape[0] // block_s,)

    chunk = 8 * hw
    n_chunks = block_s // chunk

    ccb = jnp.asarray(cc, jnp.bfloat16)
    m0f = jnp.asarray(m0)
    m1f = jnp.asarray(m1)
    lb = jnp.asarray(lmat, jnp.bfloat16)

    vmem = (2 * block_s * _L * 4 + 2 * block_s * _L * 4
            + block_s * _L * 2 // 1 + (8 << 20))

    out = pl.pallas_call(
        functools.partial(_flat_body, n_chunks=n_chunks, chunk=chunk,
                          inv_hw=inv_hw),
        out_shape=jax.ShapeDtypeStruct((rows // _L, _L), jnp.float32),
        grid=grid,
        in_specs=[
            pl.BlockSpec((block_s, _L), lambda i: (i, 0)),
            pl.BlockSpec((_L, 2 * _L), lambda i: (0, 0)),
            pl.BlockSpec((block_s, _L), lambda i: (0, 0)),
            pl.BlockSpec((block_s, _L), lambda i: (0, 0)),
            pl.BlockSpec((_GROUPS, block_s), lambda i: (0, 0)),
        ],
        out_specs=pl.BlockSpec((_GROUPS, _L), lambda i: (i, 0)),
        compiler_params=pltpu.CompilerParams(
            dimension_semantics=("parallel",),
            vmem_limit_bytes=max(vmem, 48 << 20),
        ),
        cost_estimate=pl.CostEstimate(
            flops=4 * rows * hw * _L,
            transcendentals=0,
            bytes_accessed=rows * hw * 4 + rows * 4),
    )(xf, ccb, m0f, m1f, lb)

    return out.reshape(N, C, 1, 1)


def _transpose_body(x_ref, o_ref, *, inv_hw):
    o_ref[...] = jnp.sum(x_ref[...], axis=1, keepdims=True,
                         dtype=jnp.float32) * inv_hw


def _gap_transpose(x, N, C, hw):
    inv_hw = 1.0 / float(hw)
    xt = jnp.transpose(x.reshape(N, C, hw), (0, 2, 1))
    out = pl.pallas_call(
        functools.partial(_transpose_body, inv_hw=inv_hw),
        out_shape=jax.ShapeDtypeStruct((N, 1, C), jnp.float32),
        grid=(N,),
        in_specs=[pl.BlockSpec((1, hw, C), lambda i: (i, 0, 0))],
        out_specs=pl.BlockSpec((1, 1, C), lambda i: (i, 0, 0)),
        compiler_params=pltpu.CompilerParams(
            dimension_semantics=("parallel",),
            vmem_limit_bytes=64 << 20,
        ),
    )(xt)
    return out.reshape(N, C, 1, 1)


def kernel(x):
    N, C, H, W = x.shape
    hw = H * W
    rows = N * C
    if 2 <= hw <= _L and rows % (_GROUPS * _L) == 0 and x.dtype == jnp.float32:
        return _gap_flat(x, N, C, hw)
    return _gap_transpose(x, N, C, hw)


# speedup vs baseline: 29.9744x; 29.9744x over previous
import functools

import jax
import jax.numpy as jnp
from jax.experimental import pallas as pl
from jax.experimental.pallas import tpu as pltpu


def _hwnc_body(x_ref, o_ref, *, inv_hw):
    o_ref[...] = jnp.sum(x_ref[...], axis=0, dtype=jnp.float32) * inv_hw


def _gap_hwnc(x, N, C, hw):
    inv_hw = 1.0 / float(hw)
    xp = jnp.transpose(x, (2, 3, 0, 1)).reshape(hw, N, C)

    bn = 8 if N % 8 == 0 else N
    out = pl.pallas_call(
        functools.partial(_hwnc_body, inv_hw=inv_hw),
        out_shape=jax.ShapeDtypeStruct((N, C), jnp.float32),
        grid=(N // bn,),
        in_specs=[pl.BlockSpec((hw, bn, C), lambda i: (0, i, 0))],
        out_specs=pl.BlockSpec((bn, C), lambda i: (i, 0)),
        compiler_params=pltpu.CompilerParams(
            dimension_semantics=("parallel",),
            vmem_limit_bytes=64 << 20,
        ),
        cost_estimate=pl.CostEstimate(
            flops=N * C * hw, transcendentals=0,
            bytes_accessed=N * C * hw * 4 + N * C * 4),
    )(xp)
    return out.reshape(N, C, 1, 1)


def _nhwc_body(x_ref, o_ref, *, inv_hw):
    o_ref[...] = jnp.sum(x_ref[...], axis=1, keepdims=True,
                         dtype=jnp.float32) * inv_hw


def _gap_fallback(x, N, C, hw):
    inv_hw = 1.0 / float(hw)
    xt = jnp.transpose(x.reshape(N, C, hw), (0, 2, 1))
    out = pl.pallas_call(
        functools.partial(_nhwc_body, inv_hw=inv_hw),
        out_shape=jax.ShapeDtypeStruct((N, 1, C), jnp.float32),
        grid=(N,),
        in_specs=[pl.BlockSpec((1, hw, C), lambda i: (i, 0, 0))],
        out_specs=pl.BlockSpec((1, 1, C), lambda i: (i, 0, 0)),
        compiler_params=pltpu.CompilerParams(
            dimension_semantics=("parallel",),
            vmem_limit_bytes=64 << 20,
        ),
    )(xt)
    return out.reshape(N, C, 1, 1)


def kernel(x):
    N, C, H, W = x.shape
    hw = H * W
    if C % 128 == 0 and N % 8 == 0:
        return _gap_hwnc(x, N, C, hw)
    return _gap_fallback(x, N, C, hw)


# speedup vs baseline: 33.6650x; 1.1231x over previous
import functools

import jax
import jax.numpy as jnp
from jax.experimental import pallas as pl
from jax.experimental.pallas import tpu as pltpu


def _hwnc_body(x_ref, o_ref, *, inv_hw):
    o_ref[...] = jnp.sum(x_ref[...], axis=0, dtype=jnp.float32) * inv_hw


def _gap_hwnc(x, N, C, hw):
    inv_hw = 1.0 / float(hw)
    xp = jnp.transpose(x, (2, 3, 0, 1)).reshape(hw, N, C)

    bn = 16 if N % 16 == 0 else (8 if N % 8 == 0 else N)
    out = pl.pallas_call(
        functools.partial(_hwnc_body, inv_hw=inv_hw),
        out_shape=jax.ShapeDtypeStruct((N, C), jnp.float32),
        grid=(N // bn,),
        in_specs=[pl.BlockSpec((hw, bn, C), lambda i: (0, i, 0))],
        out_specs=pl.BlockSpec((bn, C), lambda i: (i, 0)),
        compiler_params=pltpu.CompilerParams(
            dimension_semantics=("parallel",),
            vmem_limit_bytes=64 << 20,
        ),
        cost_estimate=pl.CostEstimate(
            flops=N * C * hw, transcendentals=0,
            bytes_accessed=N * C * hw * 4 + N * C * 4),
    )(xp)
    return out.reshape(N, C, 1, 1)


def _nhwc_body(x_ref, o_ref, *, inv_hw):
    o_ref[...] = jnp.sum(x_ref[...], axis=1, keepdims=True,
                         dtype=jnp.float32) * inv_hw


def _gap_fallback(x, N, C, hw):
    inv_hw = 1.0 / float(hw)
    xt = jnp.transpose(x.reshape(N, C, hw), (0, 2, 1))
    out = pl.pallas_call(
        functools.partial(_nhwc_body, inv_hw=inv_hw),
        out_shape=jax.ShapeDtypeStruct((N, 1, C), jnp.float32),
        grid=(N,),
        in_specs=[pl.BlockSpec((1, hw, C), lambda i: (i, 0, 0))],
        out_specs=pl.BlockSpec((1, 1, C), lambda i: (i, 0, 0)),
        compiler_params=pltpu.CompilerParams(
            dimension_semantics=("parallel",),
            vmem_limit_bytes=64 << 20,
        ),
    )(xt)
    return out.reshape(N, C, 1, 1)


def kernel(x):
    N, C, H, W = x.shape
    hw = H * W
    if C % 128 == 0 and N % 8 == 0:
        return _gap_hwnc(x, N, C, hw)
    return _gap_fallback(x, N, C, hw)
